# Initial kernel scaffold; baseline (speedup 1.0000x reference)
import jax
import jax.numpy as jnp
from jax.experimental import pallas as pl


def kernel(x, merge_w, merge_bn_gamma, merge_bn_beta, merge_bn_mean, merge_bn_var, cls1_w, cls1_bn_gamma, cls1_bn_beta, cls1_bn_mean, cls1_bn_var, cls2_w, cls2_bn_gamma, cls2_bn_beta, cls2_bn_mean, cls2_bn_var, reg1_w, reg1_bn_gamma, reg1_bn_beta, reg1_bn_mean, reg1_bn_var, reg2_w, reg2_bn_gamma, reg2_bn_beta, reg2_bn_mean, reg2_bn_var, cls_pred_w, cls_pred_b, reg_pred_w, reg_pred_b, obj_pred_w, obj_pred_b):
    raise NotImplementedError("write your pallas kernel here")


# speedup vs baseline: 1.2594x; 1.2594x over previous
import functools

import jax
import jax.numpy as jnp
from jax.experimental import pallas as pl
from jax.experimental.pallas import tpu as pltpu

_EPS = 1e-5


def _fused_head_kernel(Hp, Wp, margin,
                       xp_ref, mask_ref, wm_ref, bm_ref,
                       w1_ref, b1_ref, w2c_ref, b2c_ref, w2r_ref, b2r_ref,
                       whr_ref, whc_ref, bh_ref,
                       out_ref, fe, cb, rb):
    P = Hp * Wp
    R = fe.shape[0]
    C = fe.shape[1]
    mask = mask_ref[...]

    def silu_mask(y):
        s = y / (1.0 + jnp.exp(-y))
        return s * mask

    def zero_margins(ref):
        ref[pl.ds(0, margin), :] = jnp.zeros((margin, ref.shape[1]), ref.dtype)
        top = R - margin - P
        ref[pl.ds(margin + P, top), :] = jnp.zeros((top, ref.shape[1]), ref.dtype)

    zero_margins(fe)
    zero_margins(cb)
    zero_margins(rb)

    y = jnp.dot(xp_ref[0], wm_ref[...], preferred_element_type=jnp.float32)
    y = silu_mask(y + bm_ref[...])
    fe[pl.ds(margin, P), :] = y.astype(fe.dtype)

    def conv3x3(src, w_ref):
        acc = None
        for dh in range(3):
            for dw in range(3):
                off = margin + (dh - 1) * Wp + (dw - 1)
                t = jnp.dot(src[pl.ds(off, P), :], w_ref[dh * 3 + dw],
                            preferred_element_type=jnp.float32)
                acc = t if acc is None else acc + t
        return acc

    s1 = silu_mask(conv3x3(fe, w1_ref) + b1_ref[...])
    cb[pl.ds(margin, P), :] = s1[:, :C].astype(cb.dtype)
    rb[pl.ds(margin, P), :] = s1[:, C:].astype(rb.dtype)

    cls_f = silu_mask(conv3x3(cb, w2c_ref) + b2c_ref[...]).astype(jnp.bfloat16)
    reg_f = silu_mask(conv3x3(rb, w2r_ref) + b2r_ref[...]).astype(jnp.bfloat16)

    o = (jnp.dot(reg_f, whr_ref[...], preferred_element_type=jnp.float32)
         + jnp.dot(cls_f, whc_ref[...], preferred_element_type=jnp.float32)
         + bh_ref[...])
    out_ref[...] = o[None]


def _fold_bn(w_oihw, gamma, beta, mean, var):
    scale = gamma / jnp.sqrt(var + _EPS)
    return w_oihw * scale[:, None, None, None], beta - mean * scale


def _as_1x1(w_oihw):
    return jnp.transpose(w_oihw[:, :, 0, 0], (1, 0))


def _as_taps(w_oihw):
    o, i, _, _ = w_oihw.shape
    return jnp.transpose(w_oihw, (2, 3, 1, 0)).reshape(9, i, o)


def kernel(x, merge_w, merge_bn_gamma, merge_bn_beta, merge_bn_mean, merge_bn_var,
           cls1_w, cls1_bn_gamma, cls1_bn_beta, cls1_bn_mean, cls1_bn_var,
           cls2_w, cls2_bn_gamma, cls2_bn_beta, cls2_bn_mean, cls2_bn_var,
           reg1_w, reg1_bn_gamma, reg1_bn_beta, reg1_bn_mean, reg1_bn_var,
           reg2_w, reg2_bn_gamma, reg2_bn_beta, reg2_bn_mean, reg2_bn_var,
           cls_pred_w, cls_pred_b, reg_pred_w, reg_pred_b, obj_pred_w, obj_pred_b):
    n, ch, h, w = x.shape
    C = merge_w.shape[0]
    Hp, Wp = h + 2, w + 2
    P = Hp * Wp
    margin = ((Wp + 1 + 15) // 16) * 16
    R = ((margin + P + margin + 15) // 16) * 16
    bf16 = jnp.bfloat16

    xt = jnp.transpose(x, (0, 2, 3, 1)).astype(bf16)
    xp = jnp.pad(xt, ((0, 0), (1, 1), (1, 1), (0, 0))).reshape(n, P, ch)

    ar = jnp.arange(P, dtype=jnp.int32)
    hh, ww = ar // Wp, ar % Wp
    interior = ((hh >= 1) & (hh <= h) & (ww >= 1) & (ww <= w))
    mask = interior.astype(jnp.float32)[:, None]

    wm_f, bm = _fold_bn(merge_w, merge_bn_gamma, merge_bn_beta,
                        merge_bn_mean, merge_bn_var)
    wm = _as_1x1(wm_f).astype(bf16)
    w1c_f, b1c = _fold_bn(cls1_w, cls1_bn_gamma, cls1_bn_beta,
                          cls1_bn_mean, cls1_bn_var)
    w1r_f, b1r = _fold_bn(reg1_w, reg1_bn_gamma, reg1_bn_beta,
                          reg1_bn_mean, reg1_bn_var)
    w1 = jnp.concatenate([_as_taps(w1c_f), _as_taps(w1r_f)], axis=2).astype(bf16)
    b1 = jnp.concatenate([b1c, b1r])[None, :]
    w2c_f, b2c = _fold_bn(cls2_w, cls2_bn_gamma, cls2_bn_beta,
                          cls2_bn_mean, cls2_bn_var)
    w2r_f, b2r = _fold_bn(reg2_w, reg2_bn_gamma, reg2_bn_beta,
                          reg2_bn_mean, reg2_bn_var)
    w2c = _as_taps(w2c_f).astype(bf16)
    w2r = _as_taps(w2r_f).astype(bf16)

    wro = jnp.concatenate([_as_1x1(reg_pred_w), _as_1x1(obj_pred_w)], axis=1)
    wcl = _as_1x1(cls_pred_w)
    nro, ncl = wro.shape[1], wcl.shape[1]
    co = nro + ncl
    whr = jnp.concatenate([wro, jnp.zeros((C, ncl), wro.dtype)], axis=1).astype(bf16)
    whc = jnp.concatenate([jnp.zeros((C, nro), wcl.dtype), wcl], axis=1).astype(bf16)
    bh = jnp.concatenate([reg_pred_b, obj_pred_b, cls_pred_b])[None, :]

    out = pl.pallas_call(
        functools.partial(_fused_head_kernel, Hp, Wp, margin),
        out_shape=jax.ShapeDtypeStruct((n, P, co), jnp.float32),
        grid=(n,),
        in_specs=[
            pl.BlockSpec((1, P, ch), lambda i: (i, 0, 0)),
            pl.BlockSpec((P, 1), lambda i: (0, 0)),
            pl.BlockSpec((ch, C), lambda i: (0, 0)),
            pl.BlockSpec((1, C), lambda i: (0, 0)),
            pl.BlockSpec((9, C, 2 * C), lambda i: (0, 0, 0)),
            pl.BlockSpec((1, 2 * C), lambda i: (0, 0)),
            pl.BlockSpec((9, C, C), lambda i: (0, 0, 0)),
            pl.BlockSpec((1, C), lambda i: (0, 0)),
            pl.BlockSpec((9, C, C), lambda i: (0, 0, 0)),
            pl.BlockSpec((1, C), lambda i: (0, 0)),
            pl.BlockSpec((C, co), lambda i: (0, 0)),
            pl.BlockSpec((C, co), lambda i: (0, 0)),
            pl.BlockSpec((1, co), lambda i: (0, 0)),
        ],
        out_specs=pl.BlockSpec((1, P, co), lambda i: (i, 0, 0)),
        scratch_shapes=[pltpu.VMEM((R, C), bf16)] * 3,
        compiler_params=pltpu.CompilerParams(dimension_semantics=("parallel",)),
    )(xp, mask, wm, bm[None, :], w1, b1, w2c, b2c[None, :], w2r, b2r[None, :],
      whr, whc, bh)

    out = out.reshape(n, Hp, Wp, co)[:, 1:h + 1, 1:w + 1, :]
    return jnp.transpose(out, (0, 3, 1, 2))


# speedup vs baseline: 1.2757x; 1.0129x over previous
import functools

import jax
import jax.numpy as jnp
from jax.experimental import pallas as pl
from jax.experimental.pallas import tpu as pltpu

_EPS = 1e-5


def _fused_head_kernel(Hp, Wp, margin,
                       xp_ref, mask_ref, wm_ref, bm_ref,
                       w1_ref, b1_ref, w2c_ref, b2c_ref, w2r_ref, b2r_ref,
                       whr_ref, whc_ref, bh_ref,
                       out_ref, fe, cb, rb):
    P = Hp * Wp
    R = fe.shape[0]
    C = fe.shape[1]
    mask = mask_ref[...]

    def silu_mask(y):
        s = y * jax.lax.logistic(y)
        return s * mask

    def zero_margins(ref):
        ref[pl.ds(0, margin), :] = jnp.zeros((margin, ref.shape[1]), ref.dtype)
        top = R - margin - P
        ref[pl.ds(margin + P, top), :] = jnp.zeros((top, ref.shape[1]), ref.dtype)

    zero_margins(fe)
    zero_margins(cb)
    zero_margins(rb)

    y = jnp.dot(xp_ref[0], wm_ref[...], preferred_element_type=jnp.float32)
    y = silu_mask(y + bm_ref[...])
    fe[pl.ds(margin, P), :] = y.astype(fe.dtype)

    def conv3x3(src, w_ref):
        acc = None
        for dh in range(3):
            for dw in range(3):
                off = margin + (dh - 1) * Wp + (dw - 1)
                t = jnp.dot(src[pl.ds(off, P), :], w_ref[dh * 3 + dw],
                            preferred_element_type=jnp.float32)
                acc = t if acc is None else acc + t
        return acc

    acc1 = conv3x3(fe, w1_ref)
    b1 = b1_ref[...]
    cb[pl.ds(margin, P), :] = silu_mask(acc1[:, :C] + b1[:, :C]).astype(cb.dtype)
    rb[pl.ds(margin, P), :] = silu_mask(acc1[:, C:] + b1[:, C:]).astype(rb.dtype)

    cls_f = silu_mask(conv3x3(cb, w2c_ref) + b2c_ref[...]).astype(jnp.bfloat16)
    oc = jnp.dot(cls_f, whc_ref[...], preferred_element_type=jnp.float32)
    reg_f = silu_mask(conv3x3(rb, w2r_ref) + b2r_ref[...]).astype(jnp.bfloat16)
    o = oc + jnp.dot(reg_f, whr_ref[...], preferred_element_type=jnp.float32)
    out_ref[...] = (o + bh_ref[...])[None]


def _fold_bn(w_oihw, gamma, beta, mean, var):
    scale = gamma / jnp.sqrt(var + _EPS)
    return w_oihw * scale[:, None, None, None], beta - mean * scale


def _as_1x1(w_oihw):
    return jnp.transpose(w_oihw[:, :, 0, 0], (1, 0))


def _as_taps(w_oihw):
    o, i, _, _ = w_oihw.shape
    return jnp.transpose(w_oihw, (2, 3, 1, 0)).reshape(9, i, o)


def kernel(x, merge_w, merge_bn_gamma, merge_bn_beta, merge_bn_mean, merge_bn_var,
           cls1_w, cls1_bn_gamma, cls1_bn_beta, cls1_bn_mean, cls1_bn_var,
           cls2_w, cls2_bn_gamma, cls2_bn_beta, cls2_bn_mean, cls2_bn_var,
           reg1_w, reg1_bn_gamma, reg1_bn_beta, reg1_bn_mean, reg1_bn_var,
           reg2_w, reg2_bn_gamma, reg2_bn_beta, reg2_bn_mean, reg2_bn_var,
           cls_pred_w, cls_pred_b, reg_pred_w, reg_pred_b, obj_pred_w, obj_pred_b):
    n, ch, h, w = x.shape
    C = merge_w.shape[0]
    Hp, Wp = h + 2, w + 2
    P = Hp * Wp
    margin = ((Wp + 1 + 15) // 16) * 16
    R = ((margin + P + margin + 15) // 16) * 16
    bf16 = jnp.bfloat16

    xt = jnp.transpose(x, (0, 2, 3, 1)).astype(bf16)
    xp = jnp.pad(xt, ((0, 0), (1, 1), (1, 1), (0, 0))).reshape(n, P, ch)

    ar = jnp.arange(P, dtype=jnp.int32)
    hh, ww = ar // Wp, ar % Wp
    interior = ((hh >= 1) & (hh <= h) & (ww >= 1) & (ww <= w))
    mask = interior.astype(jnp.float32)[:, None]

    wm_f, bm = _fold_bn(merge_w, merge_bn_gamma, merge_bn_beta,
                        merge_bn_mean, merge_bn_var)
    wm = _as_1x1(wm_f).astype(bf16)
    w1c_f, b1c = _fold_bn(cls1_w, cls1_bn_gamma, cls1_bn_beta,
                          cls1_bn_mean, cls1_bn_var)
    w1r_f, b1r = _fold_bn(reg1_w, reg1_bn_gamma, reg1_bn_beta,
                          reg1_bn_mean, reg1_bn_var)
    w1 = jnp.concatenate([_as_taps(w1c_f), _as_taps(w1r_f)], axis=2).astype(bf16)
    b1 = jnp.concatenate([b1c, b1r])[None, :]
    w2c_f, b2c = _fold_bn(cls2_w, cls2_bn_gamma, cls2_bn_beta,
                          cls2_bn_mean, cls2_bn_var)
    w2r_f, b2r = _fold_bn(reg2_w, reg2_bn_gamma, reg2_bn_beta,
                          reg2_bn_mean, reg2_bn_var)
    w2c = _as_taps(w2c_f).astype(bf16)
    w2r = _as_taps(w2r_f).astype(bf16)

    wro = jnp.concatenate([_as_1x1(reg_pred_w), _as_1x1(obj_pred_w)], axis=1)
    wcl = _as_1x1(cls_pred_w)
    nro, ncl = wro.shape[1], wcl.shape[1]
    co = nro + ncl
    whr = jnp.concatenate([wro, jnp.zeros((C, ncl), wro.dtype)], axis=1).astype(bf16)
    whc = jnp.concatenate([jnp.zeros((C, nro), wcl.dtype), wcl], axis=1).astype(bf16)
    bh = jnp.concatenate([reg_pred_b, obj_pred_b, cls_pred_b])[None, :]

    out = pl.pallas_call(
        functools.partial(_fused_head_kernel, Hp, Wp, margin),
        out_shape=jax.ShapeDtypeStruct((n, P, co), jnp.float32),
        grid=(n,),
        in_specs=[
            pl.BlockSpec((1, P, ch), lambda i: (i, 0, 0)),
            pl.BlockSpec((P, 1), lambda i: (0, 0)),
            pl.BlockSpec((ch, C), lambda i: (0, 0)),
            pl.BlockSpec((1, C), lambda i: (0, 0)),
            pl.BlockSpec((9, C, 2 * C), lambda i: (0, 0, 0)),
            pl.BlockSpec((1, 2 * C), lambda i: (0, 0)),
            pl.BlockSpec((9, C, C), lambda i: (0, 0, 0)),
            pl.BlockSpec((1, C), lambda i: (0, 0)),
            pl.BlockSpec((9, C, C), lambda i: (0, 0, 0)),
            pl.BlockSpec((1, C), lambda i: (0, 0)),
            pl.BlockSpec((C, co), lambda i: (0, 0)),
            pl.BlockSpec((C, co), lambda i: (0, 0)),
            pl.BlockSpec((1, co), lambda i: (0, 0)),
        ],
        out_specs=pl.BlockSpec((1, P, co), lambda i: (i, 0, 0)),
        scratch_shapes=[pltpu.VMEM((R, C), bf16)] * 3,
        compiler_params=pltpu.CompilerParams(dimension_semantics=("parallel",)),
    )(xp, mask, wm, bm[None, :], w1, b1, w2c, b2c[None, :], w2r, b2r[None, :],
      whr, whc, bh)

    out = out.reshape(n, Hp, Wp, co)[:, 1:h + 1, 1:w + 1, :]
    return jnp.transpose(out, (0, 3, 1, 2))
